# Initial kernel scaffold; baseline (speedup 1.0000x reference)
import jax
import jax.numpy as jnp
from jax.experimental import pallas as pl


def kernel(x_author, x_paper, ei_writes, ei_rev, ei_cites, Wla, bla, Wlp, blp, c1_Wpa, c1_bpa, c1_Wpp, c1_bpp, c1_as_w, c1_ad_w, c1_as_r, c1_ad_r, c1_as_c, c1_ad_c, c1_Wk, c1_bk, c1_q, c2_Wpa, c2_bpa, c2_Wpp, c2_bpp, c2_as_w, c2_ad_w, c2_as_r, c2_ad_r, c2_as_c, c2_ad_c, c2_Wk, c2_bk, c2_q):
    raise NotImplementedError("write your pallas kernel here")


# speedup vs baseline: 12.4897x; 12.4897x over previous
import functools
import jax
import jax.numpy as jnp
from jax import lax
from jax.experimental import pallas as pl
from jax.experimental.pallas import tpu as pltpu
from jax.experimental.pallas import tpu_sc as plsc

N = 50000
EDGES = 800000
F = 64
HALF = 32
EB = 400
EBP = 512
NBLK = EDGES // EB
NT = 16
NPAD = 50048
ZROWS = 2 * N + 8
NTP = N + 8
NEG = -3e38


_NCH = EBP // 128


def _edge_body(s2, d, z, asrc, adst, num_o, den_o, al_o,
               sbuf, dbuf, albuf, adbuf, sbuf2, dbuf2, exbuf2, rows, stg_loc,
               stg_sh, asrc_sh, adst_sh, num_sh, den_sh, sem):
    c = lax.axis_index("c")
    t = lax.axis_index("s")
    f32 = jnp.float32

    def zrow(i, _):
        rows[i, pl.ds(0, 16)] = jnp.zeros((16,), f32)
        rows[i, pl.ds(16, 16)] = jnp.zeros((16,), f32)
        return 0
    lax.fori_loop(0, 128, zrow, 0)

    def zal(i, _):
        albuf[pl.ds(i * 16, 16)] = jnp.zeros((16,), f32)
        return 0
    lax.fori_loop(0, 8, zal, 0)

    def zchunk(k, _):
        kk = t + k * NT

        @pl.when(kk < NPAD // 128)
        def _():
            pltpu.sync_copy(rows.at[pl.ds(0, 128)],
                            num_sh.at[pl.ds(kk * 128, 128)])
            pltpu.sync_copy(albuf.at[pl.ds(0, 128)],
                            den_sh.at[pl.ds(kk * 128, 128)])
        return 0
    lax.fori_loop(0, (NPAD // 128 + NT - 1) // NT, zchunk, 0)

    for k in range((EBP - EB) // 16):
        off = EB + k * 16
        sbuf[pl.ds(off, 16)] = jnp.zeros((16,), jnp.int32)
        dbuf[pl.ds(off, 16)] = jnp.zeros((16,), jnp.int32)

    def conv_idx(src1, dst2):
        def conv(i, _):
            dst2[i // 8, pl.ds((i % 8) * 16, 16)] = src1[pl.ds(i * 16, 16)]
            return 0
        lax.fori_loop(0, EBP // 16, conv, 0)

    if True:
        @pl.when(t == 0)
        def _():
            pltpu.sync_copy(asrc.at[pl.ds(c * NTP, N)], asrc_sh)
            pltpu.sync_copy(adst.at[pl.ds(c * NTP, N)], adst_sh)
        plsc.subcore_barrier()

        def blk(bi, mx):
            base = (t + bi * NT) * EB
            pltpu.sync_copy(s2.at[pl.ds(base, EB)], sbuf.at[pl.ds(0, EB)])
            pltpu.sync_copy(d.at[pl.ds(base, EB)], dbuf.at[pl.ds(0, EB)])
            conv_idx(sbuf, sbuf2)
            conv_idx(dbuf, dbuf2)
            cps = [pltpu.async_copy(asrc_sh.at[sbuf2.at[j]],
                                    albuf.at[pl.ds(j * 128, 128)], sem)
                   for j in range(_NCH)]
            cps += [pltpu.async_copy(adst_sh.at[dbuf2.at[j]],
                                     adbuf.at[pl.ds(j * 128, 128)], sem)
                    for j in range(_NCH)]
            for cp in cps:
                cp.wait()

            def inner(i, mx):
                al = albuf[pl.ds(i * 16, 16)] + adbuf[pl.ds(i * 16, 16)]
                al = jnp.maximum(al, 0.2 * al)
                albuf[pl.ds(i * 16, 16)] = al
                return jnp.maximum(mx, al)
            mx = lax.fori_loop(0, EB // 16, inner, mx)
            pltpu.sync_copy(albuf.at[pl.ds(0, EB)],
                            al_o.at[pl.ds(c * EDGES + base, EB)])
            return mx
        mxv = lax.fori_loop(0, NBLK // NT, blk, jnp.full((16,), NEG, f32))
        adbuf[pl.ds(0, 16)] = mxv

    pltpu.sync_copy(adbuf.at[pl.ds(0, 16)], stg_sh.at[t])
    plsc.subcore_barrier()
    pltpu.sync_copy(stg_sh, stg_loc)

    def mred(i, mx):
        return jnp.maximum(mx, stg_loc[i])
    gmax = jnp.max(lax.fori_loop(0, NT, mred, jnp.full((16,), NEG, f32)))

    for k in range((EBP - EB) // 16):
        off = EB + k * 16
        sbuf[pl.ds(off, 16)] = jnp.full((16,), 2 * N, jnp.int32)
        dbuf[pl.ds(off, 16)] = jnp.full((16,), N, jnp.int32)
        albuf[pl.ds(off, 16)] = jnp.full((16,), NEG, f32)

    if True:
        def blk2(bi, _):
            base = (t + bi * NT) * EB
            pltpu.sync_copy(s2.at[pl.ds(c * EDGES + base, EB)],
                            sbuf.at[pl.ds(0, EB)])
            pltpu.sync_copy(d.at[pl.ds(base, EB)], dbuf.at[pl.ds(0, EB)])
            pltpu.sync_copy(al_o.at[pl.ds(c * EDGES + base, EB)],
                            albuf.at[pl.ds(0, EB)])
            conv_idx(sbuf, sbuf2)
            conv_idx(dbuf, dbuf2)

            def conv_ex(i, _):
                ex = jnp.exp(albuf[pl.ds(i * 16, 16)] - gmax)
                exbuf2[i // 8, pl.ds((i % 8) * 16, 16)] = ex
                return 0
            lax.fori_loop(0, EBP // 16, conv_ex, 0)

            cps = [pltpu.async_copy(z.at[sbuf2.at[j]],
                                    rows.at[pl.ds(j * 128, 128)], sem)
                   for j in range(_NCH)]
            for cp in cps:
                cp.wait()

            def srow(i, _):
                exv = exbuf2[i // 8, pl.ds((i % 8) * 16, 16)]
                for kk in range(16):
                    r = i * 16 + kk
                    e = exv[kk]
                    rows[r, pl.ds(0, 16)] = rows[r, pl.ds(0, 16)] * e
                    rows[r, pl.ds(16, 16)] = rows[r, pl.ds(16, 16)] * e
                return 0
            lax.fori_loop(0, EBP // 16, srow, 0)

            for j in range(_NCH):
                pltpu.sync_copy(rows.at[pl.ds(j * 128, 128)],
                                num_sh.at[dbuf2.at[j]], add=True)
                pltpu.sync_copy(exbuf2.at[j], den_sh.at[dbuf2.at[j]],
                                add=True)
            return 0
        lax.fori_loop(0, NBLK // NT, blk2, 0)

        plsc.subcore_barrier()

        @pl.when(t < 10)
        def _():
            r0 = t * 5000
            pltpu.sync_copy(num_sh.at[pl.ds(r0, 5000)],
                            num_o.at[c, pl.ds(r0, 5000)])
            pltpu.sync_copy(den_sh.at[pl.ds(r0, 5000)],
                            den_o.at[pl.ds(c * NTP + r0, 5000)])


_edge_sc = functools.partial(
    pl.kernel,
    out_type=(
        jax.ShapeDtypeStruct((2, N, HALF), jnp.float32),
        jax.ShapeDtypeStruct((2 * NTP,), jnp.float32),
        jax.ShapeDtypeStruct((2 * EDGES,), jnp.float32),
    ),
    mesh=plsc.VectorSubcoreMesh(core_axis_name="c", subcore_axis_name="s"),
    compiler_params=pltpu.CompilerParams(needs_layout_passes=False,
                                         use_tc_tiling_on_sc=False),
    scratch_types=[
        pltpu.VMEM((EBP,), jnp.int32),
        pltpu.VMEM((EBP,), jnp.int32),
        pltpu.VMEM((EBP,), jnp.float32),
        pltpu.VMEM((EBP,), jnp.float32),
        pltpu.VMEM((_NCH, 128), jnp.int32),
        pltpu.VMEM((_NCH, 128), jnp.int32),
        pltpu.VMEM((_NCH, 128), jnp.float32),
        pltpu.VMEM((EBP, HALF), jnp.float32),
        pltpu.VMEM((NT, 16), jnp.float32),
        pltpu.VMEM_SHARED((NT, 16), jnp.float32),
        pltpu.VMEM_SHARED((N,), jnp.float32),
        pltpu.VMEM_SHARED((N,), jnp.float32),
        pltpu.VMEM_SHARED((NPAD, HALF), jnp.float32),
        pltpu.VMEM_SHARED((NPAD,), jnp.float32),
        pltpu.SemaphoreType.DMA,
    ],
)(_edge_body)


def _edge_pass(z_flat, asrc2, adst2, s, d):
    zt = z_flat.reshape(N, 2, HALF).transpose(1, 0, 2).reshape(2 * N, HALF)
    zt = jnp.concatenate([zt, jnp.zeros((8, HALF), jnp.float32)], axis=0)
    s2 = jnp.concatenate([s, s + N]).astype(jnp.int32)
    pad8 = jnp.zeros((8,), jnp.float32)
    asf = jnp.concatenate([asrc2[0], pad8, asrc2[1], pad8])
    adf = jnp.concatenate([adst2[0], pad8, adst2[1], pad8])
    num, den, _ = _edge_sc(s2, d.astype(jnp.int32), zt, asf, adf)
    den = jnp.stack([den[:N], den[NTP:NTP + N]])
    return num, den


_R = 2000
_G = N // _R


def _k_proj_elu(x_ref, w_ref, b_ref, o_ref):
    y = jnp.dot(x_ref[...], w_ref[...],
                preferred_element_type=jnp.float32) + b_ref[...]
    o_ref[...] = jnp.where(y > 0, y, jnp.exp(y) - 1.0)


def _proj_elu(x, w, b):
    return pl.pallas_call(
        _k_proj_elu,
        grid=(_G,),
        in_specs=[pl.BlockSpec((_R, 128), lambda i: (i, 0)),
                  pl.BlockSpec((128, F), lambda i: (0, 0)),
                  pl.BlockSpec((1, F), lambda i: (0, 0))],
        out_specs=pl.BlockSpec((_R, F), lambda i: (i, 0)),
        out_shape=jax.ShapeDtypeStruct((N, F), jnp.float32),
    )(x, w, b.reshape(1, F))


def _k_proj(x_ref, w_ref, b_ref, a_ref, z_ref, al_ref):
    z = jnp.dot(x_ref[...], w_ref[...],
                preferred_element_type=jnp.float32) + b_ref[...]
    z_ref[...] = z
    al_ref[...] = jnp.dot(z, a_ref[...], preferred_element_type=jnp.float32)


def _proj(x, w, b, amat):
    return pl.pallas_call(
        _k_proj,
        grid=(_G,),
        in_specs=[pl.BlockSpec((_R, F), lambda i: (i, 0)),
                  pl.BlockSpec((F, F), lambda i: (0, 0)),
                  pl.BlockSpec((1, F), lambda i: (0, 0)),
                  pl.BlockSpec((F, 8), lambda i: (0, 0))],
        out_specs=[pl.BlockSpec((_R, F), lambda i: (i, 0)),
                   pl.BlockSpec((_R, 8), lambda i: (i, 0))],
        out_shape=[jax.ShapeDtypeStruct((N, F), jnp.float32),
                   jax.ShapeDtypeStruct((N, 8), jnp.float32)],
    )(x, w, b.reshape(1, F), amat)


def _k_fin(num_ref, den_ref, o_ref):
    n = num_ref[...]
    d0 = den_ref[:, 0:1] + 1e-16
    d1 = den_ref[:, 1:2] + 1e-16
    o_ref[...] = jnp.concatenate(
        [jnp.maximum(n[:, :HALF] / d0, 0.0),
         jnp.maximum(n[:, HALF:] / d1, 0.0)], axis=1)


def _finalize(num, den):
    num_t = num.transpose(1, 0, 2).reshape(N, F)
    den_t = den.T
    return pl.pallas_call(
        _k_fin,
        grid=(_G,),
        in_specs=[pl.BlockSpec((_R, F), lambda i: (i, 0)),
                  pl.BlockSpec((_R, 2), lambda i: (i, 0))],
        out_specs=pl.BlockSpec((_R, F), lambda i: (i, 0)),
        out_shape=jax.ShapeDtypeStruct((N, F), jnp.float32),
    )(num_t, den_t)


def _k_sem(xw_ref, xc_ref, wk_ref, bk_ref, q_ref, o_ref):
    wk = wk_ref[...]
    bk = bk_ref[...]
    q = q_ref[...]
    s0 = jnp.sum(jnp.tanh(jnp.dot(xw_ref[...], wk,
                                  preferred_element_type=jnp.float32) + bk) * q)
    s1 = jnp.sum(jnp.tanh(jnp.dot(xc_ref[...], wk,
                                  preferred_element_type=jnp.float32) + bk) * q)
    col = lax.broadcasted_iota(jnp.int32, (1, 1, 128), 2)
    o_ref[...] = jnp.where(col == 0, s0, jnp.where(col == 1, s1, 0.0))


def _k_comb(p_ref, xw_ref, xc_ref, o_ref):
    ps = jnp.sum(p_ref[...], axis=(0, 1)).reshape(1, 128) / float(N)
    s0 = ps[0:1, 0:1]
    s1 = ps[0:1, 1:2]
    m = jnp.maximum(s0, s1)
    e0 = jnp.exp(s0 - m)
    e1 = jnp.exp(s1 - m)
    w0 = e0 / (e0 + e1)
    o_ref[...] = w0 * xw_ref[...] + (1.0 - w0) * xc_ref[...]


def _semantic2(xw, xc, wk, bk, q):
    part = pl.pallas_call(
        _k_sem,
        grid=(_G,),
        in_specs=[pl.BlockSpec((_R, F), lambda i: (i, 0)),
                  pl.BlockSpec((_R, F), lambda i: (i, 0)),
                  pl.BlockSpec((F, F), lambda i: (0, 0)),
                  pl.BlockSpec((1, F), lambda i: (0, 0)),
                  pl.BlockSpec((1, F), lambda i: (0, 0))],
        out_specs=pl.BlockSpec((1, 1, 128), lambda i: (i, 0, 0)),
        out_shape=jax.ShapeDtypeStruct((_G, 1, 128), jnp.float32),
    )(xw, xc, wk, bk.reshape(1, F), q.reshape(1, F))
    return pl.pallas_call(
        _k_comb,
        grid=(_G,),
        in_specs=[pl.BlockSpec((_G, 1, 128), lambda i: (0, 0, 0)),
                  pl.BlockSpec((_R, F), lambda i: (i, 0)),
                  pl.BlockSpec((_R, F), lambda i: (i, 0))],
        out_specs=pl.BlockSpec((_R, F), lambda i: (i, 0)),
        out_shape=jax.ShapeDtypeStruct((N, F), jnp.float32),
    )(part, xw, xc)


def _amat(cols):
    a = jnp.zeros((F, 8), jnp.float32)
    for k, heads in enumerate(cols):
        nh = len(heads)
        dim = F // nh
        for h, vec in enumerate(heads):
            a = a.at[h * dim:(h + 1) * dim, 2 * k + h if nh == 2 else 2 * k].set(vec)
    return a


def _tables(al, k, nh):
    if nh == 2:
        return jnp.stack([al[:, 2 * k], al[:, 2 * k + 1]])
    col = al[:, 2 * k]
    return jnp.stack([col, col])


def _layer(x_a, x_p, ei_w, ei_r, ei_c, wpa, bpa, wpp, bpp,
           as_w, ad_w, as_r, ad_r, as_c, ad_c, nh):
    aa = _amat([[as_w[0, h] for h in range(nh)],
                [ad_r[0, h] for h in range(nh)]])
    ap = _amat([[ad_w[0, h] for h in range(nh)],
                [as_r[0, h] for h in range(nh)],
                [as_c[0, h] for h in range(nh)],
                [ad_c[0, h] for h in range(nh)]])
    za, ala = _proj(x_a, wpa, bpa, aa)
    zp, alp = _proj(x_p, wpp, bpp, ap)
    num_w, den_w = _edge_pass(za, _tables(ala, 0, nh), _tables(alp, 0, nh),
                              ei_w[0], ei_w[1])
    num_r, den_r = _edge_pass(zp, _tables(alp, 1, nh), _tables(ala, 1, nh),
                              ei_r[0], ei_r[1])
    num_c, den_c = _edge_pass(zp, _tables(alp, 2, nh), _tables(alp, 3, nh),
                              ei_c[0], ei_c[1])
    return (num_r, den_r), (num_w, den_w), (num_c, den_c)


def kernel(x_author, x_paper, ei_writes, ei_rev, ei_cites, Wla, bla, Wlp, blp,
           c1_Wpa, c1_bpa, c1_Wpp, c1_bpp, c1_as_w, c1_ad_w, c1_as_r, c1_ad_r,
           c1_as_c, c1_ad_c, c1_Wk, c1_bk, c1_q, c2_Wpa, c2_bpa, c2_Wpp,
           c2_bpp, c2_as_w, c2_ad_w, c2_as_r, c2_ad_r, c2_as_c, c2_ad_c,
           c2_Wk, c2_bk, c2_q):
    h_a = _proj_elu(x_author, Wla, bla)
    h_p = _proj_elu(x_paper, Wlp, blp)

    (nr1, dr1), (nw1, dw1), (nc1, dc1) = _layer(
        h_a, h_p, ei_writes, ei_rev, ei_cites, c1_Wpa, c1_bpa, c1_Wpp, c1_bpp,
        c1_as_w, c1_ad_w, c1_as_r, c1_ad_r, c1_as_c, c1_ad_c, 2)
    x_a = _finalize(nr1, dr1)
    xw1 = _finalize(nw1, dw1)
    xc1 = _finalize(nc1, dc1)
    x_p = _semantic2(xw1, xc1, c1_Wk, c1_bk, c1_q)

    (nr2, dr2), (nw2, dw2), (nc2, dc2) = _layer(
        x_a, x_p, ei_writes, ei_rev, ei_cites, c2_Wpa, c2_bpa, c2_Wpp, c2_bpp,
        c2_as_w, c2_ad_w, c2_as_r, c2_ad_r, c2_as_c, c2_ad_c, 1)
    out_a = _finalize(nr2, dr2)
    xw2 = _finalize(nw2, dw2)
    xc2 = _finalize(nc2, dc2)
    out_p = _semantic2(xw2, xc2, c2_Wk, c2_bk, c2_q)
    return out_a, out_p
